# baseline (speedup 1.0000x reference)
import jax
import jax.numpy as jnp
from jax.experimental import pallas as pl
from jax.experimental.pallas import tpu as pltpu


def kernel(patch, W):
    n = patch.shape[1] + 1
    d = W.shape[1]
    n_pad = (n + 7) // 8 * 8

    def body(w_hbm, o_ref, sem0):
        c0 = pltpu.make_async_copy(
            w_hbm.at[pl.ds(0, n_pad)], o_ref.at[0, pl.ds(0, n_pad)], sem0
        )
        c0.start()
        c0.wait()

    out = pl.pallas_call(
        body,
        out_shape=jax.ShapeDtypeStruct((1, n, d), W.dtype),
        in_specs=[pl.BlockSpec(memory_space=pltpu.MemorySpace.HBM)],
        out_specs=pl.BlockSpec(memory_space=pltpu.MemorySpace.HBM),
        scratch_shapes=[pltpu.SemaphoreType.DMA],
    )(W)
    return out


# speedup vs baseline: 8.2407x; 8.2407x over previous
import jax
import jax.numpy as jnp
from jax.experimental import pallas as pl
from jax.experimental.pallas import tpu as pltpu

_C0 = 296


def kernel(patch, W):
    n = patch.shape[1] + 1
    d = W.shape[1]
    n_pad = (n + 7) // 8 * 8
    c1 = n_pad - _C0

    def body(w_hbm, o_ref, s, sem_r0, sem_r1, sem_w):
        r0 = pltpu.make_async_copy(
            w_hbm.at[pl.ds(0, _C0)], s.at[pl.ds(0, _C0)], sem_r0
        )
        r1 = pltpu.make_async_copy(
            w_hbm.at[pl.ds(_C0, c1)], s.at[pl.ds(_C0, c1)], sem_r1
        )
        w = pltpu.make_async_copy(s, o_ref.at[0, pl.ds(0, n_pad)], sem_w)
        r0.start()
        r1.start()
        r0.wait()
        r1.wait()
        w.start()
        w.wait()

    out = pl.pallas_call(
        body,
        out_shape=jax.ShapeDtypeStruct((1, n, d), W.dtype),
        in_specs=[pl.BlockSpec(memory_space=pltpu.MemorySpace.HBM)],
        out_specs=pl.BlockSpec(memory_space=pltpu.MemorySpace.HBM),
        scratch_shapes=[
            pltpu.VMEM((584, 768), jnp.float32),
            pltpu.SemaphoreType.DMA,
            pltpu.SemaphoreType.DMA,
            pltpu.SemaphoreType.DMA,
        ],
    )(W)
    return out


# speedup vs baseline: 8.4826x; 1.0293x over previous
import jax
import jax.numpy as jnp
from jax.experimental import pallas as pl
from jax.experimental.pallas import tpu as pltpu

_C0 = 296


def kernel(patch, W):
    n = patch.shape[1] + 1
    d = W.shape[1]
    n_pad = (n + 7) // 8 * 8
    c1 = n_pad - _C0

    def body(w_hbm, o_ref, s0, s1, sem_r0, sem_r1, sem_w0, sem_w1):
        r0 = pltpu.make_async_copy(w_hbm.at[pl.ds(0, _C0)], s0, sem_r0)
        r1 = pltpu.make_async_copy(w_hbm.at[pl.ds(_C0, c1)], s1, sem_r1)
        tail = pl.multiple_of(jnp.int32(_C0), 8)
        w0 = pltpu.make_async_copy(s0, o_ref.at[0, pl.ds(0, _C0)], sem_w0)
        w1 = pltpu.make_async_copy(s1, o_ref.at[0, pl.ds(tail, c1)], sem_w1)
        r0.start()
        r1.start()
        r0.wait()
        w0.start()
        r1.wait()
        w1.start()
        w0.wait()
        w1.wait()

    out = pl.pallas_call(
        body,
        out_shape=jax.ShapeDtypeStruct((1, n, d), W.dtype),
        in_specs=[pl.BlockSpec(memory_space=pltpu.MemorySpace.HBM)],
        out_specs=pl.BlockSpec(memory_space=pltpu.MemorySpace.HBM),
        scratch_shapes=[
            pltpu.VMEM((_C0, 768), jnp.float32),
            pltpu.VMEM((288, 768), jnp.float32),
            pltpu.SemaphoreType.DMA,
            pltpu.SemaphoreType.DMA,
            pltpu.SemaphoreType.DMA,
            pltpu.SemaphoreType.DMA,
        ],
    )(W)
    return out


# speedup vs baseline: 8.5257x; 1.0051x over previous
import jax
import jax.numpy as jnp
from jax.experimental import pallas as pl
from jax.experimental.pallas import tpu as pltpu

_CH = (200, 192, 192)


def kernel(patch, W):
    n = patch.shape[1] + 1
    d = W.shape[1]

    def body(w_hbm, o_ref, s0, s1, s2, *sems):
        scratch = (s0, s1, s2)
        reads, writes = [], []
        off = 0
        for k, c in enumerate(_CH):
            reads.append(
                pltpu.make_async_copy(
                    w_hbm.at[pl.ds(off, c)], scratch[k], sems[k]
                )
            )
            start = off if off + c <= n else pl.multiple_of(jnp.int32(off), 8)
            writes.append(
                pltpu.make_async_copy(
                    scratch[k], o_ref.at[0, pl.ds(start, c)], sems[3 + k]
                )
            )
            off += c
        for r in reads:
            r.start()
        for k in range(len(_CH)):
            reads[k].wait()
            writes[k].start()
        for w in writes:
            w.wait()

    out = pl.pallas_call(
        body,
        out_shape=jax.ShapeDtypeStruct((1, n, d), W.dtype),
        in_specs=[pl.BlockSpec(memory_space=pltpu.MemorySpace.HBM)],
        out_specs=pl.BlockSpec(memory_space=pltpu.MemorySpace.HBM),
        scratch_shapes=[
            pltpu.VMEM((_CH[0], 768), jnp.float32),
            pltpu.VMEM((_CH[1], 768), jnp.float32),
            pltpu.VMEM((_CH[2], 768), jnp.float32),
            pltpu.SemaphoreType.DMA,
            pltpu.SemaphoreType.DMA,
            pltpu.SemaphoreType.DMA,
            pltpu.SemaphoreType.DMA,
            pltpu.SemaphoreType.DMA,
            pltpu.SemaphoreType.DMA,
        ],
    )(W)
    return out


# speedup vs baseline: 8.5321x; 1.0007x over previous
import jax
import jax.numpy as jnp
from jax.experimental import pallas as pl
from jax.experimental.pallas import tpu as pltpu

_CH = (200, 176, 144, 64)


def kernel(patch, W):
    n = patch.shape[1] + 1
    d = W.shape[1]

    def body(w_hbm, o_ref, s0, s1, s2, s3, *sems):
        scratch = (s0, s1, s2, s3)
        reads, writes = [], []
        off = 0
        for k, c in enumerate(_CH):
            reads.append(
                pltpu.make_async_copy(
                    w_hbm.at[pl.ds(off, c)], scratch[k], sems[k]
                )
            )
            start = off if off + c <= n else pl.multiple_of(jnp.int32(off), 8)
            writes.append(
                pltpu.make_async_copy(
                    scratch[k], o_ref.at[0, pl.ds(start, c)], sems[4 + k]
                )
            )
            off += c
        for r in reads:
            r.start()
        for k in range(len(_CH)):
            reads[k].wait()
            writes[k].start()
        for w in writes:
            w.wait()

    out = pl.pallas_call(
        body,
        out_shape=jax.ShapeDtypeStruct((1, n, d), W.dtype),
        in_specs=[pl.BlockSpec(memory_space=pltpu.MemorySpace.HBM)],
        out_specs=pl.BlockSpec(memory_space=pltpu.MemorySpace.HBM),
        scratch_shapes=[
            pltpu.VMEM((_CH[0], 768), jnp.float32),
            pltpu.VMEM((_CH[1], 768), jnp.float32),
            pltpu.VMEM((_CH[2], 768), jnp.float32),
            pltpu.VMEM((_CH[3], 768), jnp.float32),
        ]
        + [pltpu.SemaphoreType.DMA] * 8,
    )(W)
    return out
